# baseline (speedup 1.0000x reference)
import jax
import jax.numpy as jnp
from jax import lax
from jax.experimental import pallas as pl
from jax.experimental.pallas import tpu as pltpu
from jax.experimental.pallas import tpu_sc as plsc

N = 10000
E = 160000
NUM_GRAPHS = 128

NS = 16
L = 16
NPT = 640
N_PAD = NS * NPT
ECT = E // 128
CB = ECT // NS
EC = ECT - (NS - 1) * CB
NC = NPT // 128
NBC = N // 128
G_PAD = 256

_f32 = jnp.float32
_i32 = jnp.int32


def _rsqrt16(x):
    i = lax.bitcast_convert_type(x, _i32)
    i = jnp.int32(0x5F3759DF) - (i >> 1)
    y = lax.bitcast_convert_type(i, _f32)
    for _ in range(3):
        y = y * (1.5 - 0.5 * x * y * y)
    return y


BURST = 8


def _drain(n, drain_hbm, drain_v, sem):
    for _ in range(n):
        pltpu.make_async_copy(drain_hbm, drain_v, sem).wait()


def _edge_pass(make_src, col_v, acc, sem, ssem, n_chunks, drain_hbm, drain_v,
               tbl_s=None, row_v=None, msg_v=None):
    def grp(g, carry):
        @pl.when(g >= 2)
        def _():
            _drain(BURST, drain_hbm, drain_v, ssem)

        if tbl_s is not None:
            gds = []
            for k in range(BURST):
                j = g * BURST + k
                gds.append(
                    pltpu.async_copy(
                        tbl_s.at[row_v.at[j, 0]],
                        msg_v.at[pl.ds(j * 128, 128)],
                        sem,
                    )
                )
            for d in gds:
                d.wait()
        for k in range(BURST):
            j = g * BURST + k
            pltpu.async_copy(make_src(j), acc.at[col_v.at[j, 0]], ssem, add=True)
        return carry

    full = n_chunks // BURST if isinstance(n_chunks, int) else 72 // BURST
    lax.fori_loop(0, full, grp, 0)

    if not isinstance(n_chunks, int):
        def tail(j, carry):
            if tbl_s is not None:
                pltpu.sync_copy(
                    tbl_s.at[row_v.at[j, 0]], msg_v.at[pl.ds(j * 128, 128)]
                )
            pltpu.sync_copy(make_src(j), acc.at[col_v.at[j, 0]], add=True)
            return carry

        lax.fori_loop(full * BURST, n_chunks, tail, 0)
    _drain(2 * BURST, drain_hbm, drain_v, ssem)


def _sc_body(ei_hbm, b4_hbm, bflat_hbm, wbb_hbm, out_hbm,
             row_v, col_v, batch_v, tailb_v, msg_v,
             dis_sl, acc_sl, ds_sl, ones_v, zero_v, pg_v, wbb_v, out_v, drain_v,
             deg_s, u_s, v_s, dis_s, ds_s, tg_s, cnt_s, drain_hbm, sem, ssem):
    tid = lax.axis_index("s")
    base = tid * NPT
    nc = jnp.where(tid == NS - 1, EC, CB)

    c_row = pltpu.async_copy(ei_hbm.at[0, pl.ds(tid * CB, EC)], row_v, sem)
    c_col = pltpu.async_copy(ei_hbm.at[1, pl.ds(tid * CB, EC)], col_v, sem)
    for j in range(NC):
        @pl.when(NC * tid + j < NBC)
        def _():
            pltpu.sync_copy(b4_hbm.at[NC * tid + j], batch_v.at[j])

    @pl.when(tid == NS - 1)
    def _():
        pltpu.sync_copy(bflat_hbm.at[pl.ds(NBC * 128, L)], tailb_v)

    one16 = jnp.full((L,), 1.0, _f32)
    zero16 = jnp.zeros((L,), _f32)
    for i in range(128 // L):
        ones_v[pl.ds(i * L, L)] = one16

    def zgrp(i, carry):
        zero_v[pl.ds(i * L, L)] = zero16
        return carry

    lax.fori_loop(0, NPT // L, zgrp, 0)

    pltpu.sync_copy(zero_v, deg_s.at[pl.ds(base, NPT)])
    pltpu.sync_copy(zero_v, u_s.at[pl.ds(base, NPT)])
    pltpu.sync_copy(zero_v, v_s.at[pl.ds(base, NPT)])

    @pl.when(tid == 0)
    def _():
        pltpu.sync_copy(zero_v.at[pl.ds(0, G_PAD)], tg_s)
        pltpu.sync_copy(zero_v.at[pl.ds(0, G_PAD)], cnt_s)

    c_row.wait()
    c_col.wait()
    plsc.subcore_barrier()

    _edge_pass(lambda j: ones_v, col_v, deg_s, sem, ssem, nc, drain_hbm, drain_v)
    plsc.subcore_barrier()

    pltpu.sync_copy(deg_s.at[pl.ds(base, NPT)], acc_sl)

    def dgrp(i, carry):
        x = acc_sl[pl.ds(i * L, L)] + 1.0
        dis_sl[pl.ds(i * L, L)] = _rsqrt16(x)
        return carry

    lax.fori_loop(0, NPT // L, dgrp, 0)
    pltpu.sync_copy(dis_sl, dis_s.at[pl.ds(base, NPT)])
    plsc.subcore_barrier()

    _edge_pass(lambda j: msg_v.at[pl.ds(j * 128, 128)], col_v, u_s, sem, ssem,
               nc, drain_hbm, drain_v, tbl_s=dis_s, row_v=row_v, msg_v=msg_v)
    plsc.subcore_barrier()

    pltpu.sync_copy(u_s.at[pl.ds(base, NPT)], acc_sl)

    def sgrp(i, carry):
        d = dis_sl[pl.ds(i * L, L)]
        s = d * (acc_sl[pl.ds(i * L, L)] + d)
        ds_sl[pl.ds(i * L, L)] = d * s
        return carry

    lax.fori_loop(0, NPT // L, sgrp, 0)
    pltpu.sync_copy(ds_sl, ds_s.at[pl.ds(base, NPT)])
    plsc.subcore_barrier()

    _edge_pass(lambda j: msg_v.at[pl.ds(j * 128, 128)], col_v, v_s, sem, ssem,
               nc, drain_hbm, drain_v, tbl_s=ds_s, row_v=row_v, msg_v=msg_v)
    plsc.subcore_barrier()

    pltpu.sync_copy(v_s.at[pl.ds(base, NPT)], acc_sl)

    def tgrp(i, carry):
        t = dis_sl[pl.ds(i * L, L)] * (acc_sl[pl.ds(i * L, L)] + ds_sl[pl.ds(i * L, L)])
        acc_sl[pl.ds(i * L, L)] = t
        return carry

    lax.fori_loop(0, NPT // L, tgrp, 0)

    for j in range(NC):
        @pl.when(NC * tid + j < NBC)
        def _():
            pltpu.async_copy(
                acc_sl.at[pl.ds(j * 128, 128)], tg_s.at[batch_v.at[j, 0]],
                ssem, add=True,
            )
            pltpu.async_copy(ones_v, cnt_s.at[batch_v.at[j, 0]], ssem, add=True)

    @pl.when(tid < NS - 1)
    def _():
        _drain(2 * NC, drain_hbm, drain_v, ssem)

    @pl.when(tid == NS - 1)
    def _():
        tb = tailb_v[pl.ds(0, L)]
        loc = NBC * 128 - (NS - 1) * NPT
        pltpu.async_copy(acc_sl.at[pl.ds(loc, L)], tg_s.at[tb], ssem, add=True)
        pltpu.async_copy(ones_v.at[pl.ds(0, L)], cnt_s.at[tb], ssem, add=True)
        _drain(2 * (NBC - NC * (NS - 1)), drain_hbm, drain_v, ssem)
        for _ in range(2):
            pltpu.make_async_copy(
                drain_hbm.at[pl.ds(0, L)], drain_v.at[pl.ds(0, L)], ssem
            ).wait()
    plsc.subcore_barrier()

    @pl.when(tid == 0)
    def _():
        pltpu.sync_copy(wbb_hbm, wbb_v)
        pltpu.sync_copy(tg_s.at[pl.ds(0, 128)], pg_v.at[pl.ds(0, 128)])
        pltpu.sync_copy(cnt_s.at[pl.ds(0, 128)], pg_v.at[pl.ds(128, 128)])
        for i in range(128 // L):
            t = pg_v[pl.ds(i * L, L)]
            c = jnp.maximum(pg_v[pl.ds(128 + i * L, L)], 1.0)
            pg_v[pl.ds(i * L, L)] = t / c
        for m in range(NUM_GRAPHS * 6 // L):
            k = lax.iota(_i32, L) + (m * L)
            g = k // 6
            j = k - g * 6
            pv = plsc.load_gather(pg_v, [g])
            wv = plsc.load_gather(wbb_v, [j])
            bv = plsc.load_gather(wbb_v, [j + 8])
            out_v[pl.ds(m * L, L)] = pv * wv + bv
        pltpu.sync_copy(out_v, out_hbm)


_sc_pool = pl.kernel(
    _sc_body,
    out_type=jax.ShapeDtypeStruct((NUM_GRAPHS * 6,), _f32),
    mesh=plsc.VectorSubcoreMesh(
        core_axis_name="c", subcore_axis_name="s", num_cores=1, num_subcores=NS
    ),
    compiler_params=pltpu.CompilerParams(needs_layout_passes=False),
    scratch_types=[
        pltpu.VMEM((EC, 1, 128), _i32),
        pltpu.VMEM((EC, 1, 128), _i32),
        pltpu.VMEM((NC, 1, 128), _i32),
        pltpu.VMEM((L,), _i32),
        pltpu.VMEM((EC * 128,), _f32),
        pltpu.VMEM((NPT,), _f32),
        pltpu.VMEM((NPT,), _f32),
        pltpu.VMEM((NPT,), _f32),
        pltpu.VMEM((128,), _f32),
        pltpu.VMEM((NPT,), _f32),
        pltpu.VMEM((G_PAD,), _f32),
        pltpu.VMEM((L,), _f32),
        pltpu.VMEM((NUM_GRAPHS * 6,), _f32),
        pltpu.VMEM((128,), _f32),
        pltpu.VMEM_SHARED((N_PAD,), _f32),
        pltpu.VMEM_SHARED((N_PAD,), _f32),
        pltpu.VMEM_SHARED((N_PAD,), _f32),
        pltpu.VMEM_SHARED((N_PAD,), _f32),
        pltpu.VMEM_SHARED((N_PAD,), _f32),
        pltpu.VMEM_SHARED((G_PAD,), _f32),
        pltpu.VMEM_SHARED((G_PAD,), _f32),
        pltpu.HBM((128,), _f32),
        pltpu.SemaphoreType.DMA,
        pltpu.SemaphoreType.DMA,
    ],
)


def _dot(a, b):
    return jnp.dot(a, b, preferred_element_type=_f32, precision=lax.Precision.HIGHEST)


def _tc_body(emb_ref, w1_ref, b1_ref, w2_ref, b2_ref, wfc_ref, bfc_ref, out_ref):
    a = jnp.maximum(_dot(emb_ref[...], w1_ref[...]) + b1_ref[...], 0.0)
    g = _dot(a, w2_ref[...])
    w = _dot(g, wfc_ref[...])
    bb = _dot(b2_ref[...], wfc_ref[...]) + bfc_ref[...]
    pad2 = jnp.zeros((2,), _f32)
    out_ref[...] = jnp.concatenate([w[0], pad2, bb[0], pad2], axis=0)


def kernel(x, edge_index, batch, emb, W1, b1, W2, b2, Wfc, bfc):
    ei4 = edge_index.reshape(2, ECT, 1, 128)
    b4 = batch[: NBC * 128].reshape(NBC, 1, 128)

    wbb = pl.pallas_call(
        _tc_body,
        out_shape=jax.ShapeDtypeStruct((L,), _f32),
    )(
        emb,
        W1,
        b1.reshape(1, -1),
        W2,
        b2.reshape(1, -1),
        Wfc,
        bfc.reshape(1, -1),
    )

    out = _sc_pool(ei4, b4, batch, wbb)
    return out.reshape(NUM_GRAPHS, 6)


# speedup vs baseline: 1.0288x; 1.0288x over previous
import jax
import jax.numpy as jnp
from jax import lax
from jax.experimental import pallas as pl
from jax.experimental.pallas import tpu as pltpu
from jax.experimental.pallas import tpu_sc as plsc

N = 10000
E = 160000
NUM_GRAPHS = 128

NS = 16
L = 16
NPT = 640
N_PAD = NS * NPT
ECT = E // 128
CB = ECT // NS
EC = ECT - (NS - 1) * CB
NC = NPT // 128
G_PAD = 256

_f32 = jnp.float32
_i32 = jnp.int32


def _rsqrt16(x):
    i = lax.bitcast_convert_type(x, _i32)
    i = jnp.int32(0x5F3759DF) - (i >> 1)
    y = lax.bitcast_convert_type(i, _f32)
    for _ in range(3):
        y = y * (1.5 - 0.5 * x * y * y)
    return y


BURST = 8


def _drain(n, drain_hbm, drain_v, sem):
    for _ in range(n):
        pltpu.make_async_copy(drain_hbm, drain_v, sem).wait()


def _edge_pass(make_src, col_v, acc, sem, ssem, n_chunks, drain_hbm, drain_v,
               tbl_s=None, row_v=None, msg_v=None):
    def grp(g, carry):
        @pl.when(g >= 2)
        def _():
            _drain(BURST, drain_hbm, drain_v, ssem)

        if tbl_s is not None:
            gds = []
            for k in range(BURST):
                j = g * BURST + k
                gds.append(
                    pltpu.async_copy(
                        tbl_s.at[row_v.at[j, 0]],
                        msg_v.at[pl.ds(j * 128, 128)],
                        sem,
                    )
                )
            for d in gds:
                d.wait()
        for k in range(BURST):
            j = g * BURST + k
            pltpu.async_copy(make_src(j), acc.at[col_v.at[j, 0]], ssem, add=True)
        return carry

    full = n_chunks // BURST if isinstance(n_chunks, int) else 72 // BURST
    lax.fori_loop(0, full, grp, 0)

    if not isinstance(n_chunks, int):
        def tail(j, carry):
            if tbl_s is not None:
                pltpu.sync_copy(
                    tbl_s.at[row_v.at[j, 0]], msg_v.at[pl.ds(j * 128, 128)]
                )
            pltpu.sync_copy(make_src(j), acc.at[col_v.at[j, 0]], add=True)
            return carry

        lax.fori_loop(full * BURST, n_chunks, tail, 0)
    _drain(2 * BURST, drain_hbm, drain_v, ssem)


def _sc_body(ei_hbm, batch_hbm, wbb_hbm, out_hbm,
             row_v, col_v, batch_v, msg_v,
             dis_sl, acc_sl, ds_sl, ones_v, zero_v, pg_v, wbb_v, out_v, drain_v,
             deg_s, u_s, v_s, dis_s, ds_s, tg_s, cnt_s, drain_hbm, sem, ssem):
    tid = lax.axis_index("s")
    base = tid * NPT
    nc = jnp.where(tid == NS - 1, EC, CB)

    c_row = pltpu.async_copy(ei_hbm.at[0, pl.ds(tid * CB, EC)], row_v, sem)
    c_col = pltpu.async_copy(ei_hbm.at[1, pl.ds(tid * CB, EC)], col_v, sem)
    c_bat = pltpu.async_copy(batch_hbm.at[tid], batch_v, sem)

    one16 = jnp.full((L,), 1.0, _f32)
    zero16 = jnp.zeros((L,), _f32)
    for i in range(128 // L):
        ones_v[pl.ds(i * L, L)] = one16

    def zgrp(i, carry):
        zero_v[pl.ds(i * L, L)] = zero16
        return carry

    lax.fori_loop(0, NPT // L, zgrp, 0)

    pltpu.sync_copy(zero_v, deg_s.at[pl.ds(base, NPT)])
    pltpu.sync_copy(zero_v, u_s.at[pl.ds(base, NPT)])
    pltpu.sync_copy(zero_v, v_s.at[pl.ds(base, NPT)])

    @pl.when(tid == 0)
    def _():
        pltpu.sync_copy(zero_v.at[pl.ds(0, G_PAD)], tg_s)
        pltpu.sync_copy(zero_v.at[pl.ds(0, G_PAD)], cnt_s)

    c_row.wait()
    c_col.wait()
    c_bat.wait()
    plsc.subcore_barrier()

    _edge_pass(lambda j: ones_v, col_v, deg_s, sem, ssem, nc, drain_hbm, drain_v)
    plsc.subcore_barrier()

    pltpu.sync_copy(deg_s.at[pl.ds(base, NPT)], acc_sl)

    def dgrp(i, carry):
        x = acc_sl[pl.ds(i * L, L)] + 1.0
        dis_sl[pl.ds(i * L, L)] = _rsqrt16(x)
        return carry

    lax.fori_loop(0, NPT // L, dgrp, 0)
    pltpu.sync_copy(dis_sl, dis_s.at[pl.ds(base, NPT)])
    plsc.subcore_barrier()

    _edge_pass(lambda j: msg_v.at[pl.ds(j * 128, 128)], col_v, u_s, sem, ssem,
               nc, drain_hbm, drain_v, tbl_s=dis_s, row_v=row_v, msg_v=msg_v)
    plsc.subcore_barrier()

    pltpu.sync_copy(u_s.at[pl.ds(base, NPT)], acc_sl)

    def sgrp(i, carry):
        d = dis_sl[pl.ds(i * L, L)]
        s = d * (acc_sl[pl.ds(i * L, L)] + d)
        ds_sl[pl.ds(i * L, L)] = d * s
        return carry

    lax.fori_loop(0, NPT // L, sgrp, 0)
    pltpu.sync_copy(ds_sl, ds_s.at[pl.ds(base, NPT)])
    plsc.subcore_barrier()

    _edge_pass(lambda j: msg_v.at[pl.ds(j * 128, 128)], col_v, v_s, sem, ssem,
               nc, drain_hbm, drain_v, tbl_s=ds_s, row_v=row_v, msg_v=msg_v)
    plsc.subcore_barrier()

    pltpu.sync_copy(v_s.at[pl.ds(base, NPT)], acc_sl)

    def tgrp(i, carry):
        t = dis_sl[pl.ds(i * L, L)] * (acc_sl[pl.ds(i * L, L)] + ds_sl[pl.ds(i * L, L)])
        acc_sl[pl.ds(i * L, L)] = t
        return carry

    lax.fori_loop(0, NPT // L, tgrp, 0)

    for j in range(NC):
        pltpu.async_copy(
            acc_sl.at[pl.ds(j * 128, 128)], tg_s.at[batch_v.at[j]], ssem, add=True
        )
        pltpu.async_copy(ones_v, cnt_s.at[batch_v.at[j]], ssem, add=True)
    _drain(2 * NC, drain_hbm, drain_v, ssem)
    plsc.subcore_barrier()

    @pl.when(tid == 0)
    def _():
        pltpu.sync_copy(wbb_hbm, wbb_v)
        pltpu.sync_copy(tg_s.at[pl.ds(0, 128)], pg_v.at[pl.ds(0, 128)])
        pltpu.sync_copy(cnt_s.at[pl.ds(0, 128)], pg_v.at[pl.ds(128, 128)])
        for i in range(128 // L):
            t = pg_v[pl.ds(i * L, L)]
            c = jnp.maximum(pg_v[pl.ds(128 + i * L, L)], 1.0)
            pg_v[pl.ds(i * L, L)] = t / c
        for m in range(NUM_GRAPHS * 6 // L):
            k = lax.iota(_i32, L) + (m * L)
            g = k // 6
            j = k - g * 6
            pv = plsc.load_gather(pg_v, [g])
            wv = plsc.load_gather(wbb_v, [j])
            bv = plsc.load_gather(wbb_v, [j + 8])
            out_v[pl.ds(m * L, L)] = pv * wv + bv
        pltpu.sync_copy(out_v, out_hbm)


_sc_pool = pl.kernel(
    _sc_body,
    out_type=jax.ShapeDtypeStruct((NUM_GRAPHS * 6,), _f32),
    mesh=plsc.VectorSubcoreMesh(
        core_axis_name="c", subcore_axis_name="s", num_cores=1, num_subcores=NS
    ),
    compiler_params=pltpu.CompilerParams(needs_layout_passes=False),
    scratch_types=[
        pltpu.VMEM((EC, 1, 128), _i32),
        pltpu.VMEM((EC, 1, 128), _i32),
        pltpu.VMEM((NC, 128), _i32),
        pltpu.VMEM((EC * 128,), _f32),
        pltpu.VMEM((NPT,), _f32),
        pltpu.VMEM((NPT,), _f32),
        pltpu.VMEM((NPT,), _f32),
        pltpu.VMEM((128,), _f32),
        pltpu.VMEM((NPT,), _f32),
        pltpu.VMEM((G_PAD,), _f32),
        pltpu.VMEM((L,), _f32),
        pltpu.VMEM((NUM_GRAPHS * 6,), _f32),
        pltpu.VMEM((128,), _f32),
        pltpu.VMEM_SHARED((N_PAD,), _f32),
        pltpu.VMEM_SHARED((N_PAD,), _f32),
        pltpu.VMEM_SHARED((N_PAD,), _f32),
        pltpu.VMEM_SHARED((N_PAD,), _f32),
        pltpu.VMEM_SHARED((N_PAD,), _f32),
        pltpu.VMEM_SHARED((G_PAD,), _f32),
        pltpu.VMEM_SHARED((G_PAD,), _f32),
        pltpu.HBM((128,), _f32),
        pltpu.SemaphoreType.DMA,
        pltpu.SemaphoreType.DMA,
    ],
)


def _dot(a, b):
    return jnp.dot(a, b, preferred_element_type=_f32, precision=lax.Precision.HIGHEST)


def _tc_body(emb_ref, w1_ref, b1_ref, w2_ref, b2_ref, wfc_ref, bfc_ref, out_ref):
    a = jnp.maximum(_dot(emb_ref[...], w1_ref[...]) + b1_ref[...], 0.0)
    g = _dot(a, w2_ref[...])
    w = _dot(g, wfc_ref[...])
    bb = _dot(b2_ref[...], wfc_ref[...]) + bfc_ref[...]
    pad2 = jnp.zeros((1, 2), _f32)
    out_ref[...] = jnp.concatenate([w, pad2, bb, pad2], axis=1)


def kernel(x, edge_index, batch, emb, W1, b1, W2, b2, Wfc, bfc):
    ei4 = edge_index.reshape(2, ECT, 1, 128)
    batch_p = jnp.concatenate(
        [batch, jnp.full((N_PAD - N,), NUM_GRAPHS, _i32)]
    ).reshape(NS, NC, 128)

    wbb = pl.pallas_call(
        _tc_body,
        out_shape=jax.ShapeDtypeStruct((1, L), _f32),
    )(
        emb,
        W1,
        b1.reshape(1, -1),
        W2,
        b2.reshape(1, -1),
        Wfc,
        bfc.reshape(1, -1),
    ).reshape(L)

    out = _sc_pool(ei4, batch_p, wbb)
    return out.reshape(NUM_GRAPHS, 6)
